# Initial kernel scaffold; baseline (speedup 1.0000x reference)
import jax
import jax.numpy as jnp
from jax.experimental import pallas as pl


def kernel(x, edge_attr, params, edge_index):
    raise NotImplementedError("write your pallas kernel here")


# speedup vs baseline: 1.9736x; 1.9736x over previous
import functools

import jax
import jax.numpy as jnp
from jax import lax
from jax.experimental import pallas as pl
from jax.experimental.pallas import tpu as pltpu
from jax.experimental.pallas import tpu_sc as plsc

N_NODES = 10000
N_EDGES = 320000
NC = 2
NS = 16
NW = NC * NS
EPW = N_EDGES // NW
KCH = 80
STRIPE = 624
ZR = 208

BE = 2000
BN = 2000


def _dot(a, b):
    return lax.dot_general(a, b, (((1,), (0,)), ((), ())),
                           preferred_element_type=jnp.float32)


def _relu(v):
    return jnp.maximum(v, 0.0)


def _node_embed_body(x_ref, w0, b0, w1, b1, w2, b2, wall, ball,
                     ab_ref, c_ref):
    h = _relu(_dot(x_ref[...], w0[...]) + b0[...])
    h = _relu(_dot(h, w1[...]) + b1[...])
    node = _dot(h, w2[...]) + b2[...]
    p = _dot(node, wall[...]) + ball[...]
    ab_ref[...] = p[:, :128]
    c_ref[...] = p[:, 128:]


def _step1_body(gabd, gabs, gc, ea_ref,
                ee0w, ee0b, ee1w, ee1b, ee2w, ee2b, ee3w, ee3b,
                me0e, me1w, me1b, mn0e, e1_ref, m_ref):
    g = _relu(_dot(ea_ref[...], ee0w[...]) + ee0b[...])
    g = _relu(_dot(g, ee1w[...]) + ee1b[...])
    g = _relu(_dot(g, ee2w[...]) + ee2b[...])
    e0 = _dot(g, ee3w[...]) + ee3b[...]
    h = _relu(gabd[:, :64] + gabs[:, 64:] + _dot(e0, me0e[...]))
    e1 = _relu(_dot(h, me1w[...]) + me1b[...])
    e1_ref[...] = e1
    m_ref[...] = _relu(gc[...] + _dot(e1, mn0e[...]))


def _proj_body(parts_ref, w_ref, b_ref, ab_ref):
    node = parts_ref[0] + parts_ref[1]
    ab_ref[...] = _dot(node, w_ref[...]) + b_ref[...]


def _step2_head_body(gd, gs, e1, me0e, me1w, me1b,
                     c0w, c0b, c1w, c1b, c2w, c2b, out_ref):
    h = _relu(gd[:, :64] + gs[:, 64:] + _dot(e1[...], me0e[...]))
    e2 = _relu(_dot(h, me1w[...]) + me1b[...])
    c = _relu(_dot(e2, c0w[...]) + c0b[...])
    c = _relu(_dot(c, c1w[...]) + c1b[...])
    out_ref[...] = _dot(c, c2w[...]) + c2b[...]


def _full(shape):
    return pl.BlockSpec(shape, lambda i: (0,) * len(shape))


def _eblk(d):
    return pl.BlockSpec((BE, d), lambda i: (i, 0))


def _nblk(d):
    return pl.BlockSpec((BN, d), lambda i: (i, 0))


def _sc_mesh():
    return plsc.VectorSubcoreMesh(core_axis_name="c", subcore_axis_name="s")


def _make_gather(n_tables, pairs):
    n_out = len(pairs)

    @functools.partial(
        pl.kernel,
        out_type=tuple(jax.ShapeDtypeStruct((N_EDGES, 128), jnp.float32)
                       for _ in pairs),
        mesh=_sc_mesh(),
        scratch_types=(
            [pltpu.VMEM((KCH,), jnp.int32) for _ in range(2)]
            + [pltpu.VMEM((KCH, 128), jnp.float32) for _ in pairs]
            + [pltpu.SemaphoreType.DMA for _ in pairs]
        ),
    )
    def gather_k(*refs):
        tables = refs[:n_tables]
        dstr, srcr = refs[n_tables:n_tables + 2]
        outs = refs[n_tables + 2:n_tables + 2 + n_out]
        idxd = refs[n_tables + 2 + n_out]
        idxs = refs[n_tables + 3 + n_out]
        bufs = refs[n_tables + 4 + n_out:n_tables + 4 + 2 * n_out]
        sems = refs[n_tables + 4 + 2 * n_out:]
        idx_refs = (idxd, idxs)
        wid = lax.axis_index("s") * NC + lax.axis_index("c")

        def body(i, carry):
            base = wid * EPW + i * KCH
            pltpu.sync_copy(dstr.at[pl.ds(base, KCH)], idxd)
            pltpu.sync_copy(srcr.at[pl.ds(base, KCH)], idxs)
            cps = [pltpu.async_copy(tables[t].at[idx_refs[j]], bufs[k], sems[k])
                   for k, (t, j) in enumerate(pairs)]
            for cp in cps:
                cp.wait()
            for k in range(n_out):
                pltpu.sync_copy(bufs[k], outs[k].at[pl.ds(base, KCH)])
            return carry

        lax.fori_loop(0, EPW // KCH, body, 0)

    return gather_k


def _make_scatter():
    @functools.partial(
        pl.kernel,
        out_type=jax.ShapeDtypeStruct((NC, N_NODES, 128), jnp.float32),
        mesh=_sc_mesh(),
        scratch_types=[
            pltpu.VMEM((KCH,), jnp.int32),
            pltpu.VMEM((KCH, 128), jnp.float32),
            pltpu.VMEM((ZR, 128), jnp.float32),
            pltpu.VMEM_SHARED((N_NODES, 128), jnp.float32),
            pltpu.SemaphoreType.DMA,
        ],
    )
    def scatter_k(m, dstr, out, idx, mv, zv, acc, sem):
        cid = lax.axis_index("c")
        sid = lax.axis_index("s")
        wid = sid * NC + cid
        zero16 = jnp.zeros((16,), jnp.float32)

        def zrow(r, carry):
            for cc in range(8):
                zv[r, pl.ds(cc * 16, 16)] = zero16
            return carry

        lax.fori_loop(0, ZR, zrow, 0)
        for j in range(STRIPE // ZR):
            pltpu.sync_copy(zv, acc.at[pl.ds(sid * STRIPE + j * ZR, ZR)])

        @pl.when(sid == NS - 1)
        def _zero_tail():
            pltpu.sync_copy(zv.at[pl.ds(0, 16)],
                            acc.at[pl.ds(NS * STRIPE, 16)])

        plsc.subcore_barrier()

        def body(i, carry):
            base = wid * EPW + i * KCH
            pltpu.sync_copy(dstr.at[pl.ds(base, KCH)], idx)
            pltpu.sync_copy(m.at[pl.ds(base, KCH)], mv)
            pltpu.sync_copy(mv, acc.at[idx], add=True)
            return carry

        lax.fori_loop(0, EPW // KCH, body, 0)
        plsc.subcore_barrier()
        pltpu.sync_copy(acc.at[pl.ds(sid * STRIPE, STRIPE)],
                        out.at[cid, pl.ds(sid * STRIPE, STRIPE)])

        @pl.when(sid == NS - 1)
        def _copy_tail():
            pltpu.sync_copy(acc.at[pl.ds(NS * STRIPE, 16)],
                            out.at[cid, pl.ds(NS * STRIPE, 16)])

    return scatter_k


def kernel(x, edge_attr, params, edge_index):
    p = params
    f32 = jnp.float32
    src = edge_index[0]
    dst = edge_index[1]

    def row(v):
        return v.reshape(1, -1).astype(f32)

    wab = jnp.concatenate([p['me0_w'][:128], p['me0_w'][128:256]],
                          axis=1)
    wall = jnp.concatenate([wab, p['mn0_w'][:128]], axis=1)
    ball = jnp.concatenate([p['me0_b'], jnp.zeros((64,), f32),
                            p['mn0_b']]).reshape(1, 256)
    bab = jnp.concatenate([p['me0_b'], jnp.zeros((64,), f32)]).reshape(1, 128)
    me0e = p['me0_w'][256:272]
    mn0e = p['mn0_w'][128:144]

    ge = N_EDGES // BE
    gn = N_NODES // BN

    tab0, tc0 = pl.pallas_call(
        _node_embed_body,
        grid=(gn,),
        in_specs=[_nblk(128),
                  _full((128, 128)), _full((1, 128)),
                  _full((128, 64)), _full((1, 64)),
                  _full((64, 128)), _full((1, 128)),
                  _full((128, 256)), _full((1, 256))],
        out_specs=[_nblk(128), _nblk(128)],
        out_shape=[jax.ShapeDtypeStruct((N_NODES, 128), f32),
                   jax.ShapeDtypeStruct((N_NODES, 128), f32)],
    )(x, p['ne0_w'], row(p['ne0_b']), p['ne1_w'], row(p['ne1_b']),
      p['ne2_w'], row(p['ne2_b']), wall, ball)

    gabd, gabs, gc = _make_gather(2, [(0, 0), (0, 1), (1, 0)])(
        tab0, tc0, dst, src)

    edge1, m0 = pl.pallas_call(
        _step1_body,
        grid=(ge,),
        in_specs=[_eblk(128), _eblk(128), _eblk(128), _eblk(16),
                  _full((16, 32)), _full((1, 32)),
                  _full((32, 64)), _full((1, 64)),
                  _full((64, 64)), _full((1, 64)),
                  _full((64, 16)), _full((1, 16)),
                  _full((16, 64)), _full((64, 16)), _full((1, 16)),
                  _full((16, 128))],
        out_specs=[_eblk(16), _eblk(128)],
        out_shape=[jax.ShapeDtypeStruct((N_EDGES, 16), f32),
                   jax.ShapeDtypeStruct((N_EDGES, 128), f32)],
    )(gabd, gabs, gc, edge_attr,
      p['ee0_w'], row(p['ee0_b']), p['ee1_w'], row(p['ee1_b']),
      p['ee2_w'], row(p['ee2_b']), p['ee3_w'], row(p['ee3_b']),
      me0e, p['me1_w'], row(p['me1_b']), mn0e)

    parts = _make_scatter()(m0, dst)

    tab1 = pl.pallas_call(
        _proj_body,
        grid=(gn,),
        in_specs=[pl.BlockSpec((NC, BN, 128), lambda i: (0, i, 0)),
                  _full((128, 128)), _full((1, 128))],
        out_specs=_nblk(128),
        out_shape=jax.ShapeDtypeStruct((N_NODES, 128), f32),
    )(parts, wab, bab)

    gd1, gs1 = _make_gather(1, [(0, 0), (0, 1)])(tab1, dst, src)

    out = pl.pallas_call(
        _step2_head_body,
        grid=(ge,),
        in_specs=[_eblk(128), _eblk(128), _eblk(16),
                  _full((16, 64)), _full((64, 16)), _full((1, 16)),
                  _full((16, 64)), _full((1, 64)),
                  _full((64, 32)), _full((1, 32)),
                  _full((32, 1)), _full((1, 1))],
        out_specs=_eblk(1),
        out_shape=jax.ShapeDtypeStruct((N_EDGES, 1), f32),
    )(gd1, gs1, edge1, me0e, p['me1_w'], row(p['me1_b']),
      p['c0_w'], row(p['c0_b']), p['c1_w'], row(p['c1_b']),
      p['c2_w'], row(p['c2_b']))

    return out
